# baseline (speedup 1.0000x reference)
import functools

import jax
import jax.numpy as jnp
from jax import lax
from jax.experimental import pallas as pl
from jax.experimental.pallas import tpu as pltpu
from jax.experimental.pallas import tpu_sc as plsc

N, D, H, E, K = 2048, 1024, 2048, 8, 2
P = N * K
TILE = 64
T_MAX = P // TILE + E
S = T_MAX * TILE
T_PAD = 72
LANES = 128

NC, NSUB = 2, 16
NW = NC * NSUB
PAIRS_PER_W = P // NW
CHUNK = 32
TOK_PER_W = N // NW

_f32 = jnp.float32


def _gate_kernel(x_ref, wg_ref, bg_ref, s0_ref, s1_ref, w0_ref, w1_ref,
                 te_ref):
    x = x_ref[...]
    logits = jnp.dot(x, wg_ref[...], preferred_element_type=_f32)
    logits = logits + bg_ref[0:1, :]
    col = lax.broadcasted_iota(jnp.int32, (N, LANES), 1)
    neg = _f32(-1e30)
    logits = jnp.where(col < E, logits, neg)

    m1 = jnp.max(logits, axis=1, keepdims=True)
    i1 = jnp.min(jnp.where(logits == m1, col, LANES), axis=1, keepdims=True)
    l2 = jnp.where(col == i1, neg, logits)
    m2 = jnp.max(l2, axis=1, keepdims=True)
    i2 = jnp.min(jnp.where(l2 == m2, col, LANES), axis=1, keepdims=True)

    w0 = 1.0 / (1.0 + jnp.exp(m2 - m1))
    w1 = 1.0 - w0
    w0_ref[...] = jnp.broadcast_to(w0, (N, LANES))
    w1_ref[...] = jnp.broadcast_to(w1, (N, LANES))

    oh0 = jnp.where(col == i1, _f32(1.0), _f32(0.0))
    oh1 = jnp.where(col == i2, _f32(1.0), _f32(0.0))

    B = 256
    r = lax.broadcasted_iota(jnp.int32, (B, B), 0)
    c = lax.broadcasted_iota(jnp.int32, (B, B), 1)
    stril = jnp.where(r > c, _f32(1.0), _f32(0.0))
    carry = jnp.zeros((1, LANES), _f32)
    blocks = []
    for b in range(P // B):
        half = oh0 if b < N // B else oh1
        row0 = (b % (N // B)) * B
        blk = half[row0:row0 + B]
        blocks.append(jnp.dot(stril, blk, preferred_element_type=_f32) + carry)
        carry = carry + jnp.sum(blk, axis=0, keepdims=True)
    pos0 = jnp.concatenate(blocks[:N // B], axis=0)
    pos1 = jnp.concatenate(blocks[N // B:], axis=0)
    g = carry

    gp = jnp.ceil(g / _f32(TILE)) * _f32(TILE)
    rr = lax.broadcasted_iota(jnp.int32, (LANES, LANES), 0)
    cc = lax.broadcasted_iota(jnp.int32, (LANES, LANES), 1)
    sut = jnp.where(rr < cc, _f32(1.0), _f32(0.0))
    off = jnp.dot(gp, sut, preferred_element_type=_f32)

    s0 = jnp.sum(oh0 * (pos0 + off), axis=1, keepdims=True)
    s1 = jnp.sum(oh1 * (pos1 + off), axis=1, keepdims=True)
    s0_ref[...] = jnp.broadcast_to(s0, (N, LANES))
    s1_ref[...] = jnp.broadcast_to(s1, (N, LANES))

    end = off + gp
    trow = lax.broadcasted_iota(jnp.int32, (T_PAD, LANES), 0).astype(_f32)
    tcol = lax.broadcasted_iota(jnp.int32, (T_PAD, LANES), 1)
    hit = (jnp.broadcast_to(end, (T_PAD, LANES)) <= trow * _f32(TILE))
    cnt = jnp.sum(jnp.where(hit & (tcol < E), _f32(1.0), _f32(0.0)),
                  axis=1, keepdims=True)
    te = jnp.minimum(cnt, _f32(E - 1))
    te_ref[...] = jnp.broadcast_to(te, (T_PAD, LANES))


def _run_gate(x, wg_pad, bg_pad):
    shp = jax.ShapeDtypeStruct
    return pl.pallas_call(
        _gate_kernel,
        out_shape=(
            shp((N, LANES), _f32), shp((N, LANES), _f32),
            shp((N, LANES), _f32), shp((N, LANES), _f32),
            shp((T_PAD, LANES), _f32),
        ),
    )(x, wg_pad, bg_pad)


def _dispatch_kernel(x_hbm, wall_hbm, sidx_hbm, xs_hbm, ws_hbm,
                     idx_v, xbuf, wbuf, sem1, sem2):
    wid = lax.axis_index("s") * NC + lax.axis_index("c")
    rows_per_w = PAIRS_PER_W // CHUNK
    pltpu.sync_copy(sidx_hbm.at[pl.ds(wid * rows_per_w, rows_per_w)], idx_v)
    for c in range(rows_per_w):
        p0 = wid * PAIRS_PER_W + c * CHUNK
        tok0 = lax.rem(p0, N)
        pltpu.sync_copy(x_hbm.at[pl.ds(tok0, CHUNK)], xbuf)
        pltpu.sync_copy(wall_hbm.at[pl.ds(p0, CHUNK)], wbuf)
        cp1 = pltpu.async_copy(xbuf, xs_hbm.at[idx_v.at[c]], sem1)
        cp2 = pltpu.async_copy(wbuf, ws_hbm.at[idx_v.at[c]], sem2)
        cp1.wait()
        cp2.wait()


def _run_dispatch(x, wall, sidx):
    mesh = plsc.VectorSubcoreMesh(core_axis_name="c", subcore_axis_name="s",
                                  num_cores=NC, num_subcores=NSUB)
    shp = jax.ShapeDtypeStruct
    return pl.kernel(
        _dispatch_kernel,
        out_type=(shp((S, D), _f32), shp((S, LANES), _f32)),
        mesh=mesh,
        scratch_types=[
            pltpu.VMEM((PAIRS_PER_W // CHUNK, CHUNK), jnp.int32),
            pltpu.VMEM((CHUNK, D), _f32),
            pltpu.VMEM((CHUNK, LANES), _f32),
            pltpu.SemaphoreType.DMA,
            pltpu.SemaphoreType.DMA,
        ],
    )(x, wall, sidx)


def _mm_kernel(te_ref, xs_ref, ws_ref, w1_ref, b1_ref, w2_ref, b2_ref, y_ref):
    h = jnp.dot(xs_ref[...], w1_ref[0], preferred_element_type=_f32)
    h = jnp.maximum(h + b1_ref[0], 0.0)
    y = jnp.dot(h, w2_ref[0], preferred_element_type=_f32) + b2_ref[0]
    y_ref[...] = y * ws_ref[:, 0:1]


def _run_mm(te, xs, ws, w1, b1r, w2, b2r):
    grid_spec = pltpu.PrefetchScalarGridSpec(
        num_scalar_prefetch=1,
        grid=(T_MAX,),
        in_specs=[
            pl.BlockSpec((TILE, D), lambda t, te: (t, 0)),
            pl.BlockSpec((TILE, LANES), lambda t, te: (t, 0)),
            pl.BlockSpec((1, D, H), lambda t, te: (te[t], 0, 0)),
            pl.BlockSpec((1, 1, H), lambda t, te: (te[t], 0, 0)),
            pl.BlockSpec((1, H, D), lambda t, te: (te[t], 0, 0)),
            pl.BlockSpec((1, 1, D), lambda t, te: (te[t], 0, 0)),
        ],
        out_specs=pl.BlockSpec((TILE, D), lambda t, te: (t, 0)),
    )
    return pl.pallas_call(
        _mm_kernel,
        grid_spec=grid_spec,
        out_shape=jax.ShapeDtypeStruct((S, D), _f32),
        compiler_params=pltpu.CompilerParams(
            vmem_limit_bytes=100 * 1024 * 1024),
    )(te, xs, ws, w1, b1r, w2, b2r)


def _combine_kernel(sidx_hbm, ys_hbm, out_hbm, idx01, buf0, buf1, sem0, sem1):
    wid = lax.axis_index("s") * NC + lax.axis_index("c")
    n0 = wid * TOK_PER_W
    pltpu.sync_copy(sidx_hbm.at[:, pl.ds(wid * (TOK_PER_W // CHUNK),
                                         TOK_PER_W // CHUNK)], idx01)
    for c in range(TOK_PER_W // CHUNK):
        cp0 = pltpu.async_copy(ys_hbm.at[idx01.at[0, c]], buf0, sem0)
        cp1 = pltpu.async_copy(ys_hbm.at[idx01.at[1, c]], buf1, sem1)
        cp0.wait()
        cp1.wait()

        def add_row(r, _):
            for j in range(D // 16):
                sl = pl.ds(j * 16, 16)
                buf0[r, sl] = buf0[r, sl] + buf1[r, sl]
            return 0

        lax.fori_loop(0, CHUNK, add_row, 0)
        pltpu.sync_copy(buf0, out_hbm.at[pl.ds(n0 + c * CHUNK, CHUNK)])


def _run_combine(sidx2, ys):
    mesh = plsc.VectorSubcoreMesh(core_axis_name="c", subcore_axis_name="s",
                                  num_cores=NC, num_subcores=NSUB)
    return pl.kernel(
        _combine_kernel,
        out_type=jax.ShapeDtypeStruct((N, D), _f32),
        mesh=mesh,
        scratch_types=[
            pltpu.VMEM((2, TOK_PER_W // CHUNK, CHUNK), jnp.int32),
            pltpu.VMEM((CHUNK, D), _f32),
            pltpu.VMEM((CHUNK, D), _f32),
            pltpu.SemaphoreType.DMA,
            pltpu.SemaphoreType.DMA,
        ],
    )(sidx2, ys)


def kernel(x, Wg, bg, W1, b1, W2, b2):
    wg_pad = jnp.zeros((D, LANES), _f32).at[:, :E].set(Wg)
    bg_pad = jnp.zeros((8, LANES), _f32).at[0, :E].set(bg)

    s0f, s1f, w0b, w1b, tef = _run_gate(x, wg_pad, bg_pad)
    s0 = s0f[:, 0].astype(jnp.int32)
    s1 = s1f[:, 0].astype(jnp.int32)
    te = tef[:T_MAX, 0].astype(jnp.int32)

    sidx = jnp.concatenate([s0, s1]).reshape(P // CHUNK, CHUNK)
    wall = jnp.concatenate([w0b, w1b], axis=0)
    xs, ws = _run_dispatch(x, wall, sidx)

    ys = _run_mm(te, xs, ws, W1, b1.reshape(E, 1, H), W2, b2.reshape(E, 1, D))
    sidx2 = jnp.stack([s0, s1]).reshape(2, N // CHUNK, CHUNK)
    out = _run_combine(sidx2, ys)
    return out, 0.0


# speedup vs baseline: 1.3502x; 1.3502x over previous
import functools

import jax
import jax.numpy as jnp
from jax import lax
from jax.experimental import pallas as pl
from jax.experimental.pallas import tpu as pltpu
from jax.experimental.pallas import tpu_sc as plsc

N, D, H, E, K = 2048, 1024, 2048, 8, 2
P = N * K
TILE = 128
T_MAX = P // TILE + E
S = T_MAX * TILE
T_PAD = 48
LANES = 128

NC, NSUB = 2, 16
NW = NC * NSUB
PAIRS_PER_W = P // NW
CHUNK = 32
TOK_PER_W = N // NW

_f32 = jnp.float32


def _gate_kernel(x_ref, wg_ref, bg_ref, sall_ref, wall_ref, te_ref):
    x = x_ref[...]
    logits = jnp.dot(x, wg_ref[...], preferred_element_type=_f32)
    logits = logits + bg_ref[0:1, :]
    col = lax.broadcasted_iota(jnp.int32, (N, LANES), 1)
    neg = _f32(-1e30)
    logits = jnp.where(col < E, logits, neg)

    m1 = jnp.max(logits, axis=1, keepdims=True)
    i1 = jnp.min(jnp.where(logits == m1, col, LANES), axis=1, keepdims=True)
    l2 = jnp.where(col == i1, neg, logits)
    m2 = jnp.max(l2, axis=1, keepdims=True)
    i2 = jnp.min(jnp.where(l2 == m2, col, LANES), axis=1, keepdims=True)

    w0 = 1.0 / (1.0 + jnp.exp(m2 - m1))
    w1 = 1.0 - w0
    wall_ref[0:N, :] = jnp.broadcast_to(w0, (N, LANES))
    wall_ref[N:2 * N, :] = jnp.broadcast_to(w1, (N, LANES))

    oh0 = jnp.where(col == i1, _f32(1.0), _f32(0.0))
    oh1 = jnp.where(col == i2, _f32(1.0), _f32(0.0))

    B = 256
    r = lax.broadcasted_iota(jnp.int32, (B, B), 0)
    c = lax.broadcasted_iota(jnp.int32, (B, B), 1)
    stril = jnp.where(r > c, _f32(1.0), _f32(0.0))
    carry = jnp.zeros((1, LANES), _f32)
    blocks = []
    for b in range(P // B):
        half = oh0 if b < N // B else oh1
        row0 = (b % (N // B)) * B
        blk = half[row0:row0 + B]
        blocks.append(jnp.dot(stril, blk, preferred_element_type=_f32) + carry)
        carry = carry + jnp.sum(blk, axis=0, keepdims=True)
    pos0 = jnp.concatenate(blocks[:N // B], axis=0)
    pos1 = jnp.concatenate(blocks[N // B:], axis=0)
    g = carry

    gp = jnp.ceil(g / _f32(TILE)) * _f32(TILE)
    rr = lax.broadcasted_iota(jnp.int32, (LANES, LANES), 0)
    cc = lax.broadcasted_iota(jnp.int32, (LANES, LANES), 1)
    sut = jnp.where(rr < cc, _f32(1.0), _f32(0.0))
    off = jnp.dot(gp, sut, preferred_element_type=_f32)

    s0 = jnp.sum(oh0 * (pos0 + off), axis=1, keepdims=True)
    s1 = jnp.sum(oh1 * (pos1 + off), axis=1, keepdims=True)
    sall_ref[0:N, :] = jnp.broadcast_to(s0, (N, LANES)).astype(jnp.int32)
    sall_ref[N:2 * N, :] = jnp.broadcast_to(s1, (N, LANES)).astype(jnp.int32)

    end = off + gp
    trow = lax.broadcasted_iota(jnp.int32, (T_PAD, LANES), 0).astype(_f32)
    tcol = lax.broadcasted_iota(jnp.int32, (T_PAD, LANES), 1)
    hit = (jnp.broadcast_to(end, (T_PAD, LANES)) <= trow * _f32(TILE))
    cnt = jnp.sum(jnp.where(hit & (tcol < E), _f32(1.0), _f32(0.0)),
                  axis=1, keepdims=True)
    te = jnp.minimum(cnt, _f32(E - 1))
    te_ref[...] = jnp.broadcast_to(te, (T_PAD, LANES))


def _run_gate(x, wg_pad, bg_pad):
    shp = jax.ShapeDtypeStruct
    return pl.pallas_call(
        _gate_kernel,
        out_shape=(
            shp((P, LANES), jnp.int32), shp((P, LANES), _f32),
            shp((T_PAD, LANES), _f32),
        ),
    )(x, wg_pad, bg_pad)


def _dispatch_kernel(x_hbm, wall_hbm, sidx_hbm, xs_hbm, ws_hbm,
                     idx_v, xbuf, wbuf, sem1, sem2):
    wid = lax.axis_index("s") * NC + lax.axis_index("c")
    rows_per_w = PAIRS_PER_W // CHUNK
    pltpu.sync_copy(sidx_hbm.at[pl.ds(wid * rows_per_w, rows_per_w)], idx_v)
    for c in range(rows_per_w):
        p0 = wid * PAIRS_PER_W + c * CHUNK
        tok0 = lax.rem(p0, N)
        pltpu.sync_copy(x_hbm.at[pl.ds(tok0, CHUNK)], xbuf)
        pltpu.sync_copy(wall_hbm.at[pl.ds(p0, CHUNK)], wbuf)
        cp1 = pltpu.async_copy(xbuf, xs_hbm.at[idx_v.at[c]], sem1)
        cp2 = pltpu.async_copy(wbuf, ws_hbm.at[idx_v.at[c]], sem2)
        cp1.wait()
        cp2.wait()


def _run_dispatch(x, wall, sidx):
    mesh = plsc.VectorSubcoreMesh(core_axis_name="c", subcore_axis_name="s",
                                  num_cores=NC, num_subcores=NSUB)
    shp = jax.ShapeDtypeStruct
    return pl.kernel(
        _dispatch_kernel,
        out_type=(shp((S, D), _f32), shp((S, LANES), _f32)),
        mesh=mesh,
        scratch_types=[
            pltpu.VMEM((PAIRS_PER_W // CHUNK, CHUNK), jnp.int32),
            pltpu.VMEM((CHUNK, D), _f32),
            pltpu.VMEM((CHUNK, LANES), _f32),
            pltpu.SemaphoreType.DMA,
            pltpu.SemaphoreType.DMA,
        ],
    )(x, wall, sidx)


def _mm_kernel(te_ref, xs_ref, ws_ref, w1_ref, b1_ref, w2_ref, b2_ref, y_ref):
    h = jnp.dot(xs_ref[...], w1_ref[0], preferred_element_type=_f32)
    h = jnp.maximum(h + b1_ref[0], 0.0)
    y = jnp.dot(h, w2_ref[0], preferred_element_type=_f32) + b2_ref[0]
    y_ref[...] = y * ws_ref[:, 0:1]


def _run_mm(te, xs, ws, w1, b1r, w2, b2r):
    grid_spec = pltpu.PrefetchScalarGridSpec(
        num_scalar_prefetch=1,
        grid=(T_MAX,),
        in_specs=[
            pl.BlockSpec((TILE, D), lambda t, te: (t, 0)),
            pl.BlockSpec((TILE, LANES), lambda t, te: (t, 0)),
            pl.BlockSpec((1, D, H), lambda t, te: (te[t], 0, 0)),
            pl.BlockSpec((1, 1, H), lambda t, te: (te[t], 0, 0)),
            pl.BlockSpec((1, H, D), lambda t, te: (te[t], 0, 0)),
            pl.BlockSpec((1, 1, D), lambda t, te: (te[t], 0, 0)),
        ],
        out_specs=pl.BlockSpec((TILE, D), lambda t, te: (t, 0)),
    )
    return pl.pallas_call(
        _mm_kernel,
        grid_spec=grid_spec,
        out_shape=jax.ShapeDtypeStruct((S, D), _f32),
        compiler_params=pltpu.CompilerParams(
            vmem_limit_bytes=100 * 1024 * 1024),
    )(te, xs, ws, w1, b1r, w2, b2r)


def _combine_kernel(sidx_hbm, ys_hbm, out_hbm, idx01, buf0, buf1, sem0, sem1):
    wid = lax.axis_index("s") * NC + lax.axis_index("c")
    n0 = wid * TOK_PER_W
    pltpu.sync_copy(sidx_hbm.at[:, pl.ds(wid * (TOK_PER_W // CHUNK),
                                         TOK_PER_W // CHUNK)], idx01)
    for c in range(TOK_PER_W // CHUNK):
        cp0 = pltpu.async_copy(ys_hbm.at[idx01.at[0, c]], buf0, sem0)
        cp1 = pltpu.async_copy(ys_hbm.at[idx01.at[1, c]], buf1, sem1)
        cp0.wait()
        cp1.wait()

        def add_row(r, _):
            for j in range(D // 16):
                sl = pl.ds(j * 16, 16)
                buf0[r, sl] = buf0[r, sl] + buf1[r, sl]
            return 0

        lax.fori_loop(0, CHUNK, add_row, 0)
        pltpu.sync_copy(buf0, out_hbm.at[pl.ds(n0 + c * CHUNK, CHUNK)])


def _run_combine(sidx2, ys):
    mesh = plsc.VectorSubcoreMesh(core_axis_name="c", subcore_axis_name="s",
                                  num_cores=NC, num_subcores=NSUB)
    return pl.kernel(
        _combine_kernel,
        out_type=jax.ShapeDtypeStruct((N, D), _f32),
        mesh=mesh,
        scratch_types=[
            pltpu.VMEM((2, TOK_PER_W // CHUNK, CHUNK), jnp.int32),
            pltpu.VMEM((CHUNK, D), _f32),
            pltpu.VMEM((CHUNK, D), _f32),
            pltpu.SemaphoreType.DMA,
            pltpu.SemaphoreType.DMA,
        ],
    )(sidx2, ys)


def kernel(x, Wg, bg, W1, b1, W2, b2):
    wg_pad = jnp.zeros((D, LANES), _f32).at[:, :E].set(Wg)
    bg_pad = jnp.zeros((8, LANES), _f32).at[0, :E].set(bg)

    sall, wall, tef = _run_gate(x, wg_pad, bg_pad)
    sflat = sall[:, 0]
    te = tef[:T_MAX, 0].astype(jnp.int32)

    xs, ws = _run_dispatch(x, wall, sflat.reshape(P // CHUNK, CHUNK))
    ys = _run_mm(te, xs, ws, W1, b1.reshape(E, 1, H), W2, b2.reshape(E, 1, D))
    out = _run_combine(sflat.reshape(2, N // CHUNK, CHUNK), ys)
    return out, 0.0


# speedup vs baseline: 1.4099x; 1.0442x over previous
import functools

import jax
import jax.numpy as jnp
from jax import lax
from jax.experimental import pallas as pl
from jax.experimental.pallas import tpu as pltpu
from jax.experimental.pallas import tpu_sc as plsc

N, D, H, E, K = 2048, 1024, 2048, 8, 2
P = N * K
TILE = 128
T_MAX = P // TILE + E
S = T_MAX * TILE
T_PAD = 48
LANES = 128

NC, NSUB = 2, 16
NW = NC * NSUB
PAIRS_PER_W = P // NW
CHUNK = 32
TOK_PER_W = N // NW

_f32 = jnp.float32


def _gate_kernel(x_ref, wg_ref, bg_ref, sall_ref, wall_ref, te_ref):
    x = x_ref[...]
    logits = jnp.dot(x, wg_ref[...], preferred_element_type=_f32)
    logits = logits + bg_ref[0:1, :]
    col = lax.broadcasted_iota(jnp.int32, (N, LANES), 1)
    neg = _f32(-1e30)
    logits = jnp.where(col < E, logits, neg)

    m1 = jnp.max(logits, axis=1, keepdims=True)
    i1 = jnp.min(jnp.where(logits == m1, col, LANES), axis=1, keepdims=True)
    l2 = jnp.where(col == i1, neg, logits)
    m2 = jnp.max(l2, axis=1, keepdims=True)
    i2 = jnp.min(jnp.where(l2 == m2, col, LANES), axis=1, keepdims=True)

    w0 = 1.0 / (1.0 + jnp.exp(m2 - m1))
    w1 = 1.0 - w0
    wall_ref[0:N, :] = jnp.broadcast_to(w0, (N, LANES))
    wall_ref[N:2 * N, :] = jnp.broadcast_to(w1, (N, LANES))

    oh0 = jnp.where(col == i1, _f32(1.0), _f32(0.0))
    oh1 = jnp.where(col == i2, _f32(1.0), _f32(0.0))

    B = 256
    r = lax.broadcasted_iota(jnp.int32, (B, B), 0)
    c = lax.broadcasted_iota(jnp.int32, (B, B), 1)
    stril = jnp.where(r > c, _f32(1.0), _f32(0.0))
    carry = jnp.zeros((1, LANES), _f32)
    blocks = []
    for b in range(P // B):
        half = oh0 if b < N // B else oh1
        row0 = (b % (N // B)) * B
        blk = half[row0:row0 + B]
        blocks.append(jnp.dot(stril, blk, preferred_element_type=_f32) + carry)
        carry = carry + jnp.sum(blk, axis=0, keepdims=True)
    pos0 = jnp.concatenate(blocks[:N // B], axis=0)
    pos1 = jnp.concatenate(blocks[N // B:], axis=0)
    g = carry

    gp = jnp.ceil(g / _f32(TILE)) * _f32(TILE)
    rr = lax.broadcasted_iota(jnp.int32, (LANES, LANES), 0)
    cc = lax.broadcasted_iota(jnp.int32, (LANES, LANES), 1)
    sut = jnp.where(rr < cc, _f32(1.0), _f32(0.0))
    off = jnp.dot(gp, sut, preferred_element_type=_f32)

    s0 = jnp.sum(oh0 * (pos0 + off), axis=1, keepdims=True)
    s1 = jnp.sum(oh1 * (pos1 + off), axis=1, keepdims=True)
    sall_ref[0:N, :] = jnp.broadcast_to(s0, (N, LANES)).astype(jnp.int32)
    sall_ref[N:2 * N, :] = jnp.broadcast_to(s1, (N, LANES)).astype(jnp.int32)

    end = off + gp
    trow = lax.broadcasted_iota(jnp.int32, (T_PAD, LANES), 0).astype(_f32)
    tcol = lax.broadcasted_iota(jnp.int32, (T_PAD, LANES), 1)
    hit = (jnp.broadcast_to(end, (T_PAD, LANES)) <= trow * _f32(TILE))
    cnt = jnp.sum(jnp.where(hit & (tcol < E), _f32(1.0), _f32(0.0)),
                  axis=1, keepdims=True)
    te = jnp.minimum(cnt, _f32(E - 1))
    te_ref[...] = jnp.broadcast_to(te, (T_PAD, LANES))


def _run_gate(x, wg_pad, bg_pad):
    shp = jax.ShapeDtypeStruct
    return pl.pallas_call(
        _gate_kernel,
        out_shape=(
            shp((P, LANES), jnp.int32), shp((P, LANES), _f32),
            shp((T_PAD, LANES), _f32),
        ),
    )(x, wg_pad, bg_pad)


def _dispatch_kernel(x_hbm, wall_hbm, sidx_hbm, xs_hbm, ws_hbm,
                     idx_v, xb0, xb1, wb0, wb1, lx0, lx1, lw0, lw1,
                     sx0, sx1, sw0, sw1):
    wid = lax.axis_index("s") * NC + lax.axis_index("c")
    nch = PAIRS_PER_W // CHUNK
    pltpu.sync_copy(sidx_hbm.at[pl.ds(wid * nch, nch)], idx_v)
    xb = [xb0, xb1]
    wb = [wb0, wb1]
    lx = [lx0, lx1]
    lw = [lw0, lw1]
    sx = [sx0, sx1]
    sw = [sw0, sw1]

    def load(c):
        b = c % 2
        p0 = wid * PAIRS_PER_W + c * CHUNK
        tok0 = lax.rem(p0, N)
        return (pltpu.async_copy(x_hbm.at[pl.ds(tok0, CHUNK)], xb[b], lx[b]),
                pltpu.async_copy(wall_hbm.at[pl.ds(p0, CHUNK)], wb[b], lw[b]))

    def scat(c):
        b = c % 2
        return (pltpu.async_copy(xb[b], xs_hbm.at[idx_v.at[c]], sx[b]),
                pltpu.async_copy(wb[b], ws_hbm.at[idx_v.at[c]], sw[b]))

    loads = {0: load(0), 1: load(1)}
    scats = {}
    for c in range(nch):
        for cp in loads[c]:
            cp.wait()
        scats[c] = scat(c)
        if c + 2 < nch:
            for cp in scats[c]:
                cp.wait()
            loads[c + 2] = load(c + 2)
    for c in range(max(0, nch - 2), nch):
        for cp in scats[c]:
            cp.wait()


def _run_dispatch(x, wall, sidx):
    mesh = plsc.VectorSubcoreMesh(core_axis_name="c", subcore_axis_name="s",
                                  num_cores=NC, num_subcores=NSUB)
    shp = jax.ShapeDtypeStruct
    return pl.kernel(
        _dispatch_kernel,
        out_type=(shp((S, D), _f32), shp((S, LANES), _f32)),
        mesh=mesh,
        scratch_types=[
            pltpu.VMEM((PAIRS_PER_W // CHUNK, CHUNK), jnp.int32),
            pltpu.VMEM((CHUNK, D), _f32),
            pltpu.VMEM((CHUNK, D), _f32),
            pltpu.VMEM((CHUNK, LANES), _f32),
            pltpu.VMEM((CHUNK, LANES), _f32),
        ] + [pltpu.SemaphoreType.DMA] * 8,
    )(x, wall, sidx)


def _mm_kernel(te_ref, xs_ref, ws_ref, w1_ref, b1_ref, w2_ref, b2_ref, y_ref):
    h = jnp.dot(xs_ref[...], w1_ref[0], preferred_element_type=_f32)
    h = jnp.maximum(h + b1_ref[0], 0.0)
    y = jnp.dot(h, w2_ref[0], preferred_element_type=_f32) + b2_ref[0]
    y_ref[...] = y * ws_ref[:, 0:1]


def _run_mm(te, xs, ws, w1, b1r, w2, b2r):
    grid_spec = pltpu.PrefetchScalarGridSpec(
        num_scalar_prefetch=1,
        grid=(T_MAX,),
        in_specs=[
            pl.BlockSpec((TILE, D), lambda t, te: (t, 0)),
            pl.BlockSpec((TILE, LANES), lambda t, te: (t, 0)),
            pl.BlockSpec((1, D, H), lambda t, te: (te[t], 0, 0)),
            pl.BlockSpec((1, 1, H), lambda t, te: (te[t], 0, 0)),
            pl.BlockSpec((1, H, D), lambda t, te: (te[t], 0, 0)),
            pl.BlockSpec((1, 1, D), lambda t, te: (te[t], 0, 0)),
        ],
        out_specs=pl.BlockSpec((TILE, D), lambda t, te: (t, 0)),
    )
    return pl.pallas_call(
        _mm_kernel,
        grid_spec=grid_spec,
        out_shape=jax.ShapeDtypeStruct((S, D), _f32),
        compiler_params=pltpu.CompilerParams(
            vmem_limit_bytes=100 * 1024 * 1024),
    )(te, xs, ws, w1, b1r, w2, b2r)


CCH = 16


def _combine_kernel(sidx_hbm, ys_hbm, out_hbm, idx01, a0, a1, b0, b1,
                    g0a, g0b, g1a, g1b, ws0, ws1):
    wid = lax.axis_index("s") * NC + lax.axis_index("c")
    n0 = wid * TOK_PER_W
    nch = TOK_PER_W // CCH
    pltpu.sync_copy(sidx_hbm.at[:, pl.ds(wid * nch, nch)], idx01)
    bufs = [(a0, a1), (b0, b1)]
    gsem = [(g0a, g0b), (g1a, g1b)]
    wsem = [ws0, ws1]

    def gath(c):
        b = c % 2
        return (pltpu.async_copy(ys_hbm.at[idx01.at[0, c]], bufs[b][0],
                                 gsem[b][0]),
                pltpu.async_copy(ys_hbm.at[idx01.at[1, c]], bufs[b][1],
                                 gsem[b][1]))

    G = {0: gath(0)}
    W = {}
    for c in range(nch):
        b = c % 2
        if c + 1 < nch:
            if c - 1 >= 0:
                W[c - 1].wait()
            G[c + 1] = gath(c + 1)
        for cp in G[c]:
            cp.wait()
        dst, src = bufs[b]

        def add_row(r, _):
            for j in range(D // 16):
                sl = pl.ds(j * 16, 16)
                dst[r, sl] = dst[r, sl] + src[r, sl]
            return 0

        lax.fori_loop(0, CCH, add_row, 0)
        W[c] = pltpu.async_copy(dst, out_hbm.at[pl.ds(n0 + c * CCH, CCH)],
                                wsem[b])
    for c in range(max(0, nch - 2), nch):
        W[c].wait()


def _run_combine(sidx2, ys):
    mesh = plsc.VectorSubcoreMesh(core_axis_name="c", subcore_axis_name="s",
                                  num_cores=NC, num_subcores=NSUB)
    return pl.kernel(
        _combine_kernel,
        out_type=jax.ShapeDtypeStruct((N, D), _f32),
        mesh=mesh,
        scratch_types=[
            pltpu.VMEM((2, TOK_PER_W // CCH, CCH), jnp.int32),
            pltpu.VMEM((CCH, D), _f32),
            pltpu.VMEM((CCH, D), _f32),
            pltpu.VMEM((CCH, D), _f32),
            pltpu.VMEM((CCH, D), _f32),
        ] + [pltpu.SemaphoreType.DMA] * 6,
    )(sidx2, ys)


def kernel(x, Wg, bg, W1, b1, W2, b2):
    wg_pad = jnp.zeros((D, LANES), _f32).at[:, :E].set(Wg)
    bg_pad = jnp.zeros((8, LANES), _f32).at[0, :E].set(bg)

    sall, wall, tef = _run_gate(x, wg_pad, bg_pad)
    sflat = sall[:, 0]
    te = tef[:T_MAX, 0].astype(jnp.int32)

    xs, ws = _run_dispatch(x, wall, sflat.reshape(P // CHUNK, CHUNK))
    ys = _run_mm(te, xs, ws, W1, b1.reshape(E, 1, H), W2, b2.reshape(E, 1, D))
    out = _run_combine(sflat.reshape(2, N // CCH, CCH), ys)
    return out, 0.0


# speedup vs baseline: 1.4908x; 1.0574x over previous
import functools

import jax
import jax.numpy as jnp
from jax import lax
from jax.experimental import pallas as pl
from jax.experimental.pallas import tpu as pltpu
from jax.experimental.pallas import tpu_sc as plsc

N, D, H, E, K = 2048, 1024, 2048, 8, 2
P = N * K
TILE = 128
T_MAX = P // TILE + E
S = T_MAX * TILE
T_PAD = 48
LANES = 128

NC, NSUB = 2, 16
NW = NC * NSUB
PAIRS_PER_W = P // NW
CHUNK = 32
TOK_PER_W = N // NW

_f32 = jnp.float32


def _gate_kernel(x_ref, wg_ref, bg_ref, sall_ref, wall_ref, te_ref):
    x = x_ref[...]
    logits = jnp.dot(x, wg_ref[...], preferred_element_type=_f32)
    logits = logits + bg_ref[0:1, :]
    col = lax.broadcasted_iota(jnp.int32, (N, LANES), 1)
    neg = _f32(-1e30)
    logits = jnp.where(col < E, logits, neg)

    m1 = jnp.max(logits, axis=1, keepdims=True)
    i1 = jnp.min(jnp.where(logits == m1, col, LANES), axis=1, keepdims=True)
    l2 = jnp.where(col == i1, neg, logits)
    m2 = jnp.max(l2, axis=1, keepdims=True)
    i2 = jnp.min(jnp.where(l2 == m2, col, LANES), axis=1, keepdims=True)

    w0 = 1.0 / (1.0 + jnp.exp(m2 - m1))
    w1 = 1.0 - w0
    wall_ref[0:N, :] = jnp.broadcast_to(w0, (N, LANES))
    wall_ref[N:2 * N, :] = jnp.broadcast_to(w1, (N, LANES))

    oh0 = jnp.where(col == i1, _f32(1.0), _f32(0.0))
    oh1 = jnp.where(col == i2, _f32(1.0), _f32(0.0))

    B = 256
    r = lax.broadcasted_iota(jnp.int32, (B, B), 0)
    c = lax.broadcasted_iota(jnp.int32, (B, B), 1)
    stril = jnp.where(r > c, _f32(1.0), _f32(0.0))
    carry = jnp.zeros((1, LANES), _f32)
    blocks = []
    for b in range(P // B):
        half = oh0 if b < N // B else oh1
        row0 = (b % (N // B)) * B
        blk = half[row0:row0 + B]
        blocks.append(jnp.dot(stril, blk, preferred_element_type=_f32) + carry)
        carry = carry + jnp.sum(blk, axis=0, keepdims=True)
    pos0 = jnp.concatenate(blocks[:N // B], axis=0)
    pos1 = jnp.concatenate(blocks[N // B:], axis=0)
    g = carry

    gp = jnp.ceil(g / _f32(TILE)) * _f32(TILE)
    rr = lax.broadcasted_iota(jnp.int32, (LANES, LANES), 0)
    cc = lax.broadcasted_iota(jnp.int32, (LANES, LANES), 1)
    sut = jnp.where(rr < cc, _f32(1.0), _f32(0.0))
    off = jnp.dot(gp, sut, preferred_element_type=_f32)

    s0 = jnp.sum(oh0 * (pos0 + off), axis=1, keepdims=True)
    s1 = jnp.sum(oh1 * (pos1 + off), axis=1, keepdims=True)
    sall_ref[0:N, :] = jnp.broadcast_to(s0, (N, LANES)).astype(jnp.int32)
    sall_ref[N:2 * N, :] = jnp.broadcast_to(s1, (N, LANES)).astype(jnp.int32)

    end = off + gp
    trow = lax.broadcasted_iota(jnp.int32, (T_PAD, LANES), 0).astype(_f32)
    tcol = lax.broadcasted_iota(jnp.int32, (T_PAD, LANES), 1)
    hit = (jnp.broadcast_to(end, (T_PAD, LANES)) <= trow * _f32(TILE))
    cnt = jnp.sum(jnp.where(hit & (tcol < E), _f32(1.0), _f32(0.0)),
                  axis=1, keepdims=True)
    te = jnp.minimum(cnt, _f32(E - 1))
    te_ref[...] = jnp.broadcast_to(te, (T_PAD, LANES))


def _run_gate(x, wg_pad, bg_pad):
    shp = jax.ShapeDtypeStruct
    return pl.pallas_call(
        _gate_kernel,
        out_shape=(
            shp((P, LANES), jnp.int32), shp((P, LANES), _f32),
            shp((T_PAD, LANES), _f32),
        ),
    )(x, wg_pad, bg_pad)


def _dispatch_kernel(x_hbm, wall_hbm, sidx_hbm, xs_hbm, ws_hbm,
                     idx_v, xb0, xb1, wb0, wb1, lx0, lx1, lw0, lw1,
                     sx0, sx1, sw0, sw1):
    wid = lax.axis_index("s") * NC + lax.axis_index("c")
    nch = PAIRS_PER_W // CHUNK
    pltpu.sync_copy(sidx_hbm.at[pl.ds(wid * nch, nch)], idx_v)
    xb = [xb0, xb1]
    wb = [wb0, wb1]
    lx = [lx0, lx1]
    lw = [lw0, lw1]
    sx = [sx0, sx1]
    sw = [sw0, sw1]

    def load(c):
        b = c % 2
        p0 = wid * PAIRS_PER_W + c * CHUNK
        tok0 = lax.rem(p0, N)
        return (pltpu.async_copy(x_hbm.at[pl.ds(tok0, CHUNK)], xb[b], lx[b]),
                pltpu.async_copy(wall_hbm.at[pl.ds(p0, CHUNK)], wb[b], lw[b]))

    def scat(c):
        b = c % 2
        return (pltpu.async_copy(xb[b], xs_hbm.at[idx_v.at[c]], sx[b]),
                pltpu.async_copy(wb[b], ws_hbm.at[idx_v.at[c]], sw[b]))

    loads = {0: load(0), 1: load(1)}
    scats = {}
    for c in range(nch):
        for cp in loads[c]:
            cp.wait()
        scats[c] = scat(c)
        if c + 2 < nch:
            for cp in scats[c]:
                cp.wait()
            loads[c + 2] = load(c + 2)
    for c in range(max(0, nch - 2), nch):
        for cp in scats[c]:
            cp.wait()


def _run_dispatch(x, wall, sidx):
    mesh = plsc.VectorSubcoreMesh(core_axis_name="c", subcore_axis_name="s",
                                  num_cores=NC, num_subcores=NSUB)
    shp = jax.ShapeDtypeStruct
    return pl.kernel(
        _dispatch_kernel,
        out_type=(shp((S, D), _f32), shp((S, LANES), _f32)),
        mesh=mesh,
        scratch_types=[
            pltpu.VMEM((PAIRS_PER_W // CHUNK, CHUNK), jnp.int32),
            pltpu.VMEM((CHUNK, D), _f32),
            pltpu.VMEM((CHUNK, D), _f32),
            pltpu.VMEM((CHUNK, LANES), _f32),
            pltpu.VMEM((CHUNK, LANES), _f32),
        ] + [pltpu.SemaphoreType.DMA] * 8,
    )(x, wall, sidx)


def _mm_kernel(m_ref, xs_ref, ws_ref, b1_ref, b2_ref, w1_hbm, w2_hbm, y_ref,
               w1b0, w1b1, w2b0, w2b1, semA, semB):
    t = pl.program_id(0)
    e = m_ref[0, t]
    nxt = m_ref[1, t]
    cb = m_ref[2, t]
    st = m_ref[3, t]

    @pl.when(t == 0)
    def _():
        pltpu.make_async_copy(w1_hbm.at[e], w1b0, semA).start()
        pltpu.make_async_copy(w2_hbm.at[e], w2b0, semA).start()

    @pl.when((st == 1) & (nxt >= 0) & (cb == 0))
    def _():
        pltpu.make_async_copy(w1_hbm.at[nxt], w1b1, semB).start()
        pltpu.make_async_copy(w2_hbm.at[nxt], w2b1, semB).start()

    @pl.when((st == 1) & (nxt >= 0) & (cb == 1))
    def _():
        pltpu.make_async_copy(w1_hbm.at[nxt], w1b0, semA).start()
        pltpu.make_async_copy(w2_hbm.at[nxt], w2b0, semA).start()

    @pl.when((st == 1) & (cb == 0))
    def _():
        pltpu.make_async_copy(w1_hbm.at[0], w1b0, semA).wait()
        pltpu.make_async_copy(w2_hbm.at[0], w2b0, semA).wait()

    @pl.when((st == 1) & (cb == 1))
    def _():
        pltpu.make_async_copy(w1_hbm.at[0], w1b1, semB).wait()
        pltpu.make_async_copy(w2_hbm.at[0], w2b1, semB).wait()

    def compute(w1b, w2b):
        h = jnp.dot(xs_ref[...], w1b[...], preferred_element_type=_f32)
        h = jnp.maximum(h + b1_ref[0], 0.0)
        y = jnp.dot(h, w2b[...], preferred_element_type=_f32) + b2_ref[0]
        y_ref[...] = y * ws_ref[:, 0:1]

    @pl.when(cb == 0)
    def _():
        compute(w1b0, w2b0)

    @pl.when(cb == 1)
    def _():
        compute(w1b1, w2b1)


def _run_mm(meta, xs, ws, w1, b1r, w2, b2r):
    grid_spec = pltpu.PrefetchScalarGridSpec(
        num_scalar_prefetch=1,
        grid=(T_MAX,),
        in_specs=[
            pl.BlockSpec((TILE, D), lambda t, m: (t, 0)),
            pl.BlockSpec((TILE, LANES), lambda t, m: (t, 0)),
            pl.BlockSpec((1, 1, H), lambda t, m: (m[0, t], 0, 0)),
            pl.BlockSpec((1, 1, D), lambda t, m: (m[0, t], 0, 0)),
            pl.BlockSpec(memory_space=pl.ANY),
            pl.BlockSpec(memory_space=pl.ANY),
        ],
        out_specs=pl.BlockSpec((TILE, D), lambda t, m: (t, 0)),
        scratch_shapes=[
            pltpu.VMEM((D, H), _f32),
            pltpu.VMEM((D, H), _f32),
            pltpu.VMEM((H, D), _f32),
            pltpu.VMEM((H, D), _f32),
            pltpu.SemaphoreType.DMA,
            pltpu.SemaphoreType.DMA,
        ],
    )
    return pl.pallas_call(
        _mm_kernel,
        grid_spec=grid_spec,
        out_shape=jax.ShapeDtypeStruct((S, D), _f32),
        compiler_params=pltpu.CompilerParams(
            vmem_limit_bytes=100 * 1024 * 1024),
    )(meta, xs, ws, b1r, b2r, w1, w2)


CCH = 16


def _combine_kernel(sidx_hbm, ys_hbm, out_hbm, idx01, a0, a1, b0, b1,
                    g0a, g0b, g1a, g1b, ws0, ws1):
    wid = lax.axis_index("s") * NC + lax.axis_index("c")
    n0 = wid * TOK_PER_W
    nch = TOK_PER_W // CCH
    pltpu.sync_copy(sidx_hbm.at[:, pl.ds(wid * nch, nch)], idx01)
    bufs = [(a0, a1), (b0, b1)]
    gsem = [(g0a, g0b), (g1a, g1b)]
    wsem = [ws0, ws1]

    def gath(c):
        b = c % 2
        return (pltpu.async_copy(ys_hbm.at[idx01.at[0, c]], bufs[b][0],
                                 gsem[b][0]),
                pltpu.async_copy(ys_hbm.at[idx01.at[1, c]], bufs[b][1],
                                 gsem[b][1]))

    G = {0: gath(0)}
    W = {}
    for c in range(nch):
        b = c % 2
        if c + 1 < nch:
            if c - 1 >= 0:
                W[c - 1].wait()
            G[c + 1] = gath(c + 1)
        for cp in G[c]:
            cp.wait()
        dst, src = bufs[b]

        def add_row(r, _):
            for j in range(D // 16):
                sl = pl.ds(j * 16, 16)
                dst[r, sl] = dst[r, sl] + src[r, sl]
            return 0

        lax.fori_loop(0, CCH, add_row, 0)
        W[c] = pltpu.async_copy(dst, out_hbm.at[pl.ds(n0 + c * CCH, CCH)],
                                wsem[b])
    for c in range(max(0, nch - 2), nch):
        W[c].wait()


def _run_combine(sidx2, ys):
    mesh = plsc.VectorSubcoreMesh(core_axis_name="c", subcore_axis_name="s",
                                  num_cores=NC, num_subcores=NSUB)
    return pl.kernel(
        _combine_kernel,
        out_type=jax.ShapeDtypeStruct((N, D), _f32),
        mesh=mesh,
        scratch_types=[
            pltpu.VMEM((2, TOK_PER_W // CCH, CCH), jnp.int32),
            pltpu.VMEM((CCH, D), _f32),
            pltpu.VMEM((CCH, D), _f32),
            pltpu.VMEM((CCH, D), _f32),
            pltpu.VMEM((CCH, D), _f32),
        ] + [pltpu.SemaphoreType.DMA] * 6,
    )(sidx2, ys)


def kernel(x, Wg, bg, W1, b1, W2, b2):
    wg_pad = jnp.zeros((D, LANES), _f32).at[:, :E].set(Wg)
    bg_pad = jnp.zeros((8, LANES), _f32).at[0, :E].set(bg)

    sall, wall, tef = _run_gate(x, wg_pad, bg_pad)
    sflat = sall[:, 0]
    te = tef[:T_MAX, 0].astype(jnp.int32)

    start = jnp.concatenate([jnp.ones((1,), bool), te[1:] != te[:-1]])
    rid = jnp.cumsum(start.astype(jnp.int32)) - 1
    re = jnp.full((T_MAX + 1,), -1, jnp.int32).at[rid].set(te)
    nxt = jnp.where(start, re[jnp.clip(rid + 1, 0, T_MAX)], -1)
    meta = jnp.stack([te, nxt, rid % 2, start.astype(jnp.int32)])

    xs, ws = _run_dispatch(x, wall, sflat.reshape(P // CHUNK, CHUNK))
    ys = _run_mm(meta, xs, ws, W1, b1.reshape(E, 1, H), W2,
                 b2.reshape(E, 1, D))
    out = _run_combine(sflat.reshape(2, N // CCH, CCH), ys)
    return out, 0.0
